# baseline (speedup 1.0000x reference)
import functools

import jax
import jax.numpy as jnp
from jax import lax
from jax.experimental import pallas as pl
from jax.experimental.pallas import tpu as pltpu
from jax.experimental.pallas import tpu_sc as plsc

B = 16384
ENT_DIM = 64
REL_DIM = 64
OUT_DIM = ENT_DIM + REL_DIM + ENT_DIM
PAIR_W = 2 * ENT_DIM

NC = 2
NS = 16
NW = NC * NS
ROWS_PER_W = B // NW
CHUNK = 128
N_CHUNKS = ROWS_PER_W // CHUNK
L = 16

_GATHER_DNUMS = lax.GatherDimensionNumbers(
    offset_dims=(), collapsed_slice_dims=(0,), start_index_map=(0,))


def _lane_shuffle(v, idx):
    return lax.gather(v, idx[:, None], _GATHER_DNUMS, (1,),
                      mode=lax.GatherScatterMode.PROMISE_IN_BOUNDS)


def _rsqrt_newton(x):
    i = lax.bitcast_convert_type(x, jnp.int32)
    i = jnp.int32(0x5F3759DF) - lax.shift_right_arithmetic(i, jnp.int32(1))
    y = lax.bitcast_convert_type(i, jnp.float32)
    half_x = x * jnp.float32(0.5)
    for _ in range(2):
        y = y * (jnp.float32(1.5) - half_x * y * y)
    return y


def _body(h_hbm, r_hbm, t_hbm, ent_hbm, rel_hbm, out_hbm,
          idx_h, idx_r, idx_t, pidx_h, pidx_r, pidx_t,
          rows_h, rows_r, rows_t, out_v, sem):
    wid = lax.axis_index("s") * NC + lax.axis_index("c")
    w_base = wid * ROWS_PER_W

    def chunk_body(ci, _):
        base = w_base + ci * CHUNK
        pltpu.sync_copy(h_hbm.at[pl.ds(base, CHUNK)], idx_h)
        pltpu.sync_copy(r_hbm.at[pl.ds(base, CHUNK)], idx_r)
        pltpu.sync_copy(t_hbm.at[pl.ds(base, CHUNK)], idx_t)

        def halve(g, _):
            s = pl.ds(g * L, L)
            pidx_h[s] = lax.shift_right_logical(idx_h[s], 1)
            pidx_r[s] = lax.shift_right_logical(idx_r[s], 1)
            pidx_t[s] = lax.shift_right_logical(idx_t[s], 1)
            return 0

        lax.fori_loop(0, CHUNK // L, halve, 0)

        cp_h = pltpu.async_copy(ent_hbm.at[pidx_h], rows_h, sem)
        cp_r = pltpu.async_copy(rel_hbm.at[pidx_r], rows_r, sem)
        cp_t = pltpu.async_copy(ent_hbm.at[pidx_t], rows_t, sem)
        cp_h.wait()
        cp_r.wait()
        cp_t.wait()

        lanes = lax.iota(jnp.int32, L)

        @plsc.parallel_loop(0, CHUNK // L, step=1, unroll=2)
        def group_body(g):
            gbase = g * L
            ivs_h = idx_h[pl.ds(gbase, L)]
            ivs_r = idx_r[pl.ds(gbase, L)]
            ivs_t = idx_t[pl.ds(gbase, L)]
            for k in range(L):
                i = gbase + k
                off_h = (ivs_h[k] & jnp.int32(1)) * jnp.int32(ENT_DIM)
                off_r = (ivs_r[k] & jnp.int32(1)) * jnp.int32(ENT_DIM)
                off_t = (ivs_t[k] & jnp.int32(1)) * jnp.int32(ENT_DIM)
                xs = []
                acc = jnp.zeros((L,), jnp.float32)
                for src, off in ((rows_h, off_h), (rows_r, off_r),
                                 (rows_t, off_t)):
                    for c in range(ENT_DIM // L):
                        x = src[i, pl.ds(off + c * L, L)]
                        xs.append(x)
                        acc = acc + x * x
                for s in (8, 4, 2, 1):
                    acc = acc + _lane_shuffle(acc, lanes ^ s)
                invv = _rsqrt_newton(jnp.maximum(acc, jnp.float32(1e-24)))
                for c, x in enumerate(xs):
                    out_v[i, pl.ds(c * L, L)] = x * invv

        pltpu.sync_copy(out_v, out_hbm.at[pl.ds(base, CHUNK)])
        return 0

    lax.fori_loop(0, N_CHUNKS, chunk_body, 0)


def kernel(h, r, t, ent_weight, rel_weight):
    k = functools.partial(
        pl.kernel,
        out_type=jax.ShapeDtypeStruct((B, OUT_DIM), jnp.float32),
        mesh=plsc.VectorSubcoreMesh(core_axis_name="c", subcore_axis_name="s"),
        compiler_params=pltpu.CompilerParams(use_tc_tiling_on_sc=True),
        scratch_types=[
            pltpu.VMEM((CHUNK,), jnp.int32),
            pltpu.VMEM((CHUNK,), jnp.int32),
            pltpu.VMEM((CHUNK,), jnp.int32),
            pltpu.VMEM((CHUNK,), jnp.int32),
            pltpu.VMEM((CHUNK,), jnp.int32),
            pltpu.VMEM((CHUNK,), jnp.int32),
            pltpu.VMEM((CHUNK, PAIR_W), jnp.float32),
            pltpu.VMEM((CHUNK, PAIR_W), jnp.float32),
            pltpu.VMEM((CHUNK, PAIR_W), jnp.float32),
            pltpu.VMEM((CHUNK, OUT_DIM), jnp.float32),
            pltpu.SemaphoreType.DMA,
        ],
    )(_body)
    ent2 = ent_weight.reshape(ent_weight.shape[0] // 2, PAIR_W)
    rel2 = rel_weight.reshape(rel_weight.shape[0] // 2, PAIR_W)
    return k(h.astype(jnp.int32), r.astype(jnp.int32), t.astype(jnp.int32),
             ent2, rel2)


# speedup vs baseline: 1.5384x; 1.5384x over previous
import functools

import jax
import jax.numpy as jnp
from jax import lax
from jax.experimental import pallas as pl
from jax.experimental.pallas import tpu as pltpu
from jax.experimental.pallas import tpu_sc as plsc

B = 16384
ENT_DIM = 64
REL_DIM = 64
N_REL = 1000
OUT_DIM = ENT_DIM + REL_DIM + ENT_DIM

NC = 2
NS = 16
NW = NC * NS
ROWS_PER_W = B // NW
CHUNK = 64
N_CHUNKS = ROWS_PER_W // CHUNK
L = 16

_GATHER_DNUMS = lax.GatherDimensionNumbers(
    offset_dims=(), collapsed_slice_dims=(0,), start_index_map=(0,))


def _lane_shuffle(v, idx):
    return lax.gather(v, idx[:, None], _GATHER_DNUMS, (1,),
                      mode=lax.GatherScatterMode.PROMISE_IN_BOUNDS)


def _rsqrt_newton(x):
    i = lax.bitcast_convert_type(x, jnp.int32)
    i = jnp.int32(0x5F3759DF) - lax.shift_right_arithmetic(i, jnp.int32(1))
    y = lax.bitcast_convert_type(i, jnp.float32)
    half_x = x * jnp.float32(0.5)
    for _ in range(2):
        y = y * (jnp.float32(1.5) - half_x * y * y)
    return y


def _body(h_hbm, r_hbm, t_hbm, ent_hbm, relT_hbm, out_hbm,
          idx_h0, idx_h1, idx_t0, idx_t1, idx_r,
          rows_h0, rows_h1, rows_t0, rows_t1, rel_v, out_v,
          sem0, sem1, semr):
    wid = lax.axis_index("s") * NC + lax.axis_index("c")
    w_base = wid * ROWS_PER_W

    pltpu.sync_copy(relT_hbm, rel_v)

    idx_h = (idx_h0, idx_h1)
    idx_t = (idx_t0, idx_t1)
    rows_h = (rows_h0, rows_h1)
    rows_t = (rows_t0, rows_t1)
    sems = (sem0, sem1)

    def stage(ci, p):
        base = w_base + ci * CHUNK
        pltpu.sync_copy(h_hbm.at[pl.ds(base, CHUNK)], idx_h[p])
        pltpu.sync_copy(t_hbm.at[pl.ds(base, CHUNK)], idx_t[p])

        def fire(g, _):
            gbase = g * L
            ivs_h = idx_h[p][pl.ds(gbase, L)]
            ivs_t = idx_t[p][pl.ds(gbase, L)]
            for k in range(L):
                j = gbase + k
                pltpu.async_copy(ent_hbm.at[pl.ds(ivs_h[k], 1)],
                                 rows_h[p].at[pl.ds(j, 1)], sems[p])
                pltpu.async_copy(ent_hbm.at[pl.ds(ivs_t[k], 1)],
                                 rows_t[p].at[pl.ds(j, 1)], sems[p])
            return 0

        lax.fori_loop(0, CHUNK // L, fire, 0)

    def compute(ci, p):
        base = w_base + ci * CHUNK
        pltpu.sync_copy(r_hbm.at[pl.ds(base, CHUNK)], idx_r)
        pltpu.make_async_copy(ent_hbm.at[pl.ds(0, CHUNK)], rows_h[p],
                              sems[p]).wait()
        pltpu.make_async_copy(ent_hbm.at[pl.ds(0, CHUNK)], rows_t[p],
                              sems[p]).wait()

        lanes = lax.iota(jnp.int32, L)

        @plsc.parallel_loop(0, CHUNK // L, step=1, unroll=2)
        def group_body(g):
            gbase = g * L
            ivs_r = idx_r[pl.ds(gbase, L)]
            for k in range(L):
                j = gbase + k
                rj = ivs_r[k]
                xs = []
                acc = jnp.zeros((L,), jnp.float32)
                for src in (rows_h[p], rows_t[p]):
                    for c in range(ENT_DIM // L):
                        x = src[j, pl.ds(c * L, L)]
                        xs.append(x)
                        acc = acc + x * x
                rel_base = rj * jnp.int32(REL_DIM)
                for c in range(REL_DIM // L):
                    x = rel_v[pl.ds(rel_base + c * L, L)]
                    xs.append(x)
                    acc = acc + x * x
                for s in (8, 4, 2, 1):
                    acc = acc + _lane_shuffle(acc, lanes ^ s)
                invv = _rsqrt_newton(jnp.maximum(acc, jnp.float32(1e-24)))
                for c in range(ENT_DIM // L):
                    out_v[j, pl.ds(c * L, L)] = xs[c] * invv
                for c in range(REL_DIM // L):
                    out_v[j, pl.ds(ENT_DIM + c * L, L)] = xs[8 + c] * invv
                for c in range(ENT_DIM // L):
                    out_v[j, pl.ds(ENT_DIM + REL_DIM + c * L, L)] = \
                        xs[4 + c] * invv

        pltpu.sync_copy(out_v, out_hbm.at[pl.ds(base, CHUNK)])

    stage(0, 0)

    def chunk_pair(d, _):
        ci = d * 2

        @pl.when(ci + 1 < N_CHUNKS)
        def _():
            stage(ci + 1, 1)

        compute(ci, 0)

        @pl.when(ci + 2 < N_CHUNKS)
        def _():
            stage(ci + 2, 0)

        @pl.when(ci + 1 < N_CHUNKS)
        def _():
            compute(ci + 1, 1)

        return 0

    lax.fori_loop(0, (N_CHUNKS + 1) // 2, chunk_pair, 0)


def kernel(h, r, t, ent_weight, rel_weight):
    k = functools.partial(
        pl.kernel,
        out_type=jax.ShapeDtypeStruct((B, OUT_DIM), jnp.float32),
        mesh=plsc.VectorSubcoreMesh(core_axis_name="c", subcore_axis_name="s"),
        compiler_params=pltpu.CompilerParams(use_tc_tiling_on_sc=True),
        scratch_types=[
            pltpu.VMEM((CHUNK,), jnp.int32),
            pltpu.VMEM((CHUNK,), jnp.int32),
            pltpu.VMEM((CHUNK,), jnp.int32),
            pltpu.VMEM((CHUNK,), jnp.int32),
            pltpu.VMEM((CHUNK,), jnp.int32),
            pltpu.VMEM((CHUNK, ENT_DIM), jnp.float32),
            pltpu.VMEM((CHUNK, ENT_DIM), jnp.float32),
            pltpu.VMEM((CHUNK, ENT_DIM), jnp.float32),
            pltpu.VMEM((CHUNK, ENT_DIM), jnp.float32),
            pltpu.VMEM((REL_DIM * N_REL,), jnp.float32),
            pltpu.VMEM((CHUNK, OUT_DIM), jnp.float32),
            pltpu.SemaphoreType.DMA,
            pltpu.SemaphoreType.DMA,
            pltpu.SemaphoreType.DMA,
        ],
    )(_body)
    return k(h.astype(jnp.int32), r.astype(jnp.int32), t.astype(jnp.int32),
             ent_weight, rel_weight.reshape(-1))


# speedup vs baseline: 1.6481x; 1.0713x over previous
import functools

import jax
import jax.numpy as jnp
from jax import lax
from jax.experimental import pallas as pl
from jax.experimental.pallas import tpu as pltpu
from jax.experimental.pallas import tpu_sc as plsc

B = 16384
ENT_DIM = 64
REL_DIM = 64
N_REL = 1000
OUT_DIM = ENT_DIM + REL_DIM + ENT_DIM

NC = 2
NS = 16
NW = NC * NS
ROWS_PER_W = B // NW
CHUNK = 128
N_CHUNKS = ROWS_PER_W // CHUNK
L = 16

_GATHER_DNUMS = lax.GatherDimensionNumbers(
    offset_dims=(), collapsed_slice_dims=(0,), start_index_map=(0,))


def _lane_shuffle(v, idx):
    return lax.gather(v, idx[:, None], _GATHER_DNUMS, (1,),
                      mode=lax.GatherScatterMode.PROMISE_IN_BOUNDS)


def _rsqrt_newton(x):
    i = lax.bitcast_convert_type(x, jnp.int32)
    i = jnp.int32(0x5F3759DF) - lax.shift_right_arithmetic(i, jnp.int32(1))
    y = lax.bitcast_convert_type(i, jnp.float32)
    half_x = x * jnp.float32(0.5)
    for _ in range(2):
        y = y * (jnp.float32(1.5) - half_x * y * y)
    return y


def _body(h_hbm, r_hbm, t_hbm, ent_hbm, rel_hbm, out_hbm,
          idx_h, idx_r, idx_t, rows_h, rows_t, rel_v, out_v, sem):
    wid = lax.axis_index("s") * NC + lax.axis_index("c")
    w_base = wid * ROWS_PER_W

    pltpu.sync_copy(rel_hbm, rel_v)

    def chunk_body(ci, _):
        base = w_base + ci * CHUNK
        pltpu.sync_copy(h_hbm.at[pl.ds(base, CHUNK)], idx_h)
        pltpu.sync_copy(t_hbm.at[pl.ds(base, CHUNK)], idx_t)
        pltpu.sync_copy(r_hbm.at[pl.ds(base, CHUNK)], idx_r)

        def fire(g, _):
            gbase = g * L
            ivs_h = idx_h[pl.ds(gbase, L)]
            ivs_t = idx_t[pl.ds(gbase, L)]
            for k in range(L):
                j = gbase + k
                pltpu.async_copy(ent_hbm.at[pl.ds(ivs_h[k], 1)],
                                 rows_h.at[pl.ds(j, 1)], sem)
                pltpu.async_copy(ent_hbm.at[pl.ds(ivs_t[k], 1)],
                                 rows_t.at[pl.ds(j, 1)], sem)
            return 0

        lax.fori_loop(0, CHUNK // L, fire, 0)
        pltpu.make_async_copy(ent_hbm.at[pl.ds(0, CHUNK)], rows_h, sem).wait()
        pltpu.make_async_copy(ent_hbm.at[pl.ds(0, CHUNK)], rows_t, sem).wait()

        lanes = lax.iota(jnp.int32, L)

        @plsc.parallel_loop(0, CHUNK // L, step=1)
        def group_body(g):
            gbase = g * L
            ivs_r = idx_r[pl.ds(gbase, L)]
            for k in range(L):
                j = gbase + k
                rel_base = ivs_r[k] * jnp.int32(REL_DIM)
                xs = []
                acc = jnp.zeros((L,), jnp.float32)
                for src in (rows_h, rows_t):
                    for c in range(ENT_DIM // L):
                        x = src[j, pl.ds(c * L, L)]
                        xs.append(x)
                        acc = acc + x * x
                for c in range(REL_DIM // L):
                    x = rel_v[pl.ds(rel_base + c * L, L)]
                    xs.append(x)
                    acc = acc + x * x
                for s in (8, 4, 2, 1):
                    acc = acc + _lane_shuffle(acc, lanes ^ s)
                invv = _rsqrt_newton(jnp.maximum(acc, jnp.float32(1e-24)))
                for c in range(ENT_DIM // L):
                    out_v[j, pl.ds(c * L, L)] = xs[c] * invv
                for c in range(REL_DIM // L):
                    out_v[j, pl.ds(ENT_DIM + c * L, L)] = xs[8 + c] * invv
                for c in range(ENT_DIM // L):
                    out_v[j, pl.ds(ENT_DIM + REL_DIM + c * L, L)] = \
                        xs[4 + c] * invv

        pltpu.sync_copy(out_v, out_hbm.at[pl.ds(base, CHUNK)])
        return 0

    lax.fori_loop(0, N_CHUNKS, chunk_body, 0)


def kernel(h, r, t, ent_weight, rel_weight):
    k = functools.partial(
        pl.kernel,
        out_type=jax.ShapeDtypeStruct((B, OUT_DIM), jnp.float32),
        mesh=plsc.VectorSubcoreMesh(core_axis_name="c", subcore_axis_name="s"),
        compiler_params=pltpu.CompilerParams(use_tc_tiling_on_sc=True),
        scratch_types=[
            pltpu.VMEM((CHUNK,), jnp.int32),
            pltpu.VMEM((CHUNK,), jnp.int32),
            pltpu.VMEM((CHUNK,), jnp.int32),
            pltpu.VMEM((CHUNK, ENT_DIM), jnp.float32),
            pltpu.VMEM((CHUNK, ENT_DIM), jnp.float32),
            pltpu.VMEM((REL_DIM * N_REL,), jnp.float32),
            pltpu.VMEM((CHUNK, OUT_DIM), jnp.float32),
            pltpu.SemaphoreType.DMA,
        ],
    )(_body)
    return k(h.astype(jnp.int32), r.astype(jnp.int32), t.astype(jnp.int32),
             ent_weight, rel_weight.reshape(-1))


# speedup vs baseline: 1.9933x; 1.2094x over previous
import functools

import jax
import jax.numpy as jnp
from jax import lax
from jax.experimental import pallas as pl
from jax.experimental.pallas import tpu as pltpu
from jax.experimental.pallas import tpu_sc as plsc

B = 16384
N_ENT = 1000000
ENT_DIM = 64
REL_DIM = 64
N_REL = 1000
OUT_DIM = ENT_DIM + REL_DIM + ENT_DIM

NC = 2
NS = 16
NW = NC * NS
ROWS_PER_W = B // NW
CHUNK = 128
N_CHUNKS = ROWS_PER_W // CHUNK
L = 16

TR_BR = 8192
TR_STEPS = -(-N_ENT // TR_BR)


def _tc_transpose_body(x_ref, o_ref):
    o_ref[...] = x_ref[...].T


def _tc_transpose(entT):
    return pl.pallas_call(
        _tc_transpose_body,
        grid=(TR_STEPS,),
        in_specs=[pl.BlockSpec((ENT_DIM, TR_BR), lambda i: (0, i))],
        out_specs=pl.BlockSpec((TR_BR, ENT_DIM), lambda i: (i, 0)),
        out_shape=jax.ShapeDtypeStruct((N_ENT, ENT_DIM), jnp.float32),
    )(entT)


def _rsqrt_newton(x):
    i = lax.bitcast_convert_type(x, jnp.int32)
    i = jnp.int32(0x5F3759DF) - lax.shift_right_arithmetic(i, jnp.int32(1))
    y = lax.bitcast_convert_type(i, jnp.float32)
    half_x = x * jnp.float32(0.5)
    for _ in range(2):
        y = y * (jnp.float32(1.5) - half_x * y * y)
    return y


_GATHER_DNUMS = lax.GatherDimensionNumbers(
    offset_dims=(), collapsed_slice_dims=(0,), start_index_map=(0,))


def _lane_shuffle(v, idx):
    return lax.gather(v, idx[:, None], _GATHER_DNUMS, (1,),
                      mode=lax.GatherScatterMode.PROMISE_IN_BOUNDS)


def _body(h_hbm, r_hbm, t_hbm, ent_hbm, rel_hbm, out_hbm,
          idx_h, idx_r, idx_t, rows_h, rows_t, rel_v, out_v, sem):
    wid = lax.axis_index("s") * NC + lax.axis_index("c")
    w_base = wid * ROWS_PER_W

    pltpu.sync_copy(rel_hbm, rel_v)

    def chunk_body(ci, _):
        base = w_base + ci * CHUNK
        pltpu.sync_copy(h_hbm.at[pl.ds(base, CHUNK)], idx_h)
        pltpu.sync_copy(t_hbm.at[pl.ds(base, CHUNK)], idx_t)
        pltpu.sync_copy(r_hbm.at[pl.ds(base, CHUNK)], idx_r)

        def fire(g, _):
            gbase = g * L
            ivs_h = idx_h[pl.ds(gbase, L)]
            ivs_t = idx_t[pl.ds(gbase, L)]
            for k in range(L):
                j = gbase + k
                pltpu.async_copy(ent_hbm.at[pl.ds(ivs_h[k], 1)],
                                 rows_h.at[pl.ds(j, 1)], sem)
                pltpu.async_copy(ent_hbm.at[pl.ds(ivs_t[k], 1)],
                                 rows_t.at[pl.ds(j, 1)], sem)
            return 0

        lax.fori_loop(0, CHUNK // L, fire, 0)
        pltpu.make_async_copy(ent_hbm.at[pl.ds(0, CHUNK)], rows_h, sem).wait()
        pltpu.make_async_copy(ent_hbm.at[pl.ds(0, CHUNK)], rows_t, sem).wait()

        lanes = lax.iota(jnp.int32, L)

        @plsc.parallel_loop(0, CHUNK // L, step=1)
        def group_body(g):
            gbase = g * L
            ivs_r = idx_r[pl.ds(gbase, L)]
            for k in range(L):
                j = gbase + k
                rel_base = ivs_r[k] * jnp.int32(REL_DIM)
                xs = []
                acc = jnp.zeros((L,), jnp.float32)
                for src in (rows_h, rows_t):
                    for c in range(ENT_DIM // L):
                        x = src[j, pl.ds(c * L, L)]
                        xs.append(x)
                        acc = acc + x * x
                for c in range(REL_DIM // L):
                    x = rel_v[pl.ds(rel_base + c * L, L)]
                    xs.append(x)
                    acc = acc + x * x
                for s in (8, 4, 2, 1):
                    acc = acc + _lane_shuffle(acc, lanes ^ s)
                invv = _rsqrt_newton(jnp.maximum(acc, jnp.float32(1e-24)))
                for c in range(ENT_DIM // L):
                    out_v[j, pl.ds(c * L, L)] = xs[c] * invv
                for c in range(REL_DIM // L):
                    out_v[j, pl.ds(ENT_DIM + c * L, L)] = xs[8 + c] * invv
                for c in range(ENT_DIM // L):
                    out_v[j, pl.ds(ENT_DIM + REL_DIM + c * L, L)] = \
                        xs[4 + c] * invv

        pltpu.sync_copy(out_v, out_hbm.at[pl.ds(base, CHUNK)])
        return 0

    lax.fori_loop(0, N_CHUNKS, chunk_body, 0)


def kernel(h, r, t, ent_weight, rel_weight):
    ent_rm = _tc_transpose(ent_weight.T)
    k = functools.partial(
        pl.kernel,
        out_type=jax.ShapeDtypeStruct((B, OUT_DIM), jnp.float32),
        mesh=plsc.VectorSubcoreMesh(core_axis_name="c", subcore_axis_name="s"),
        compiler_params=pltpu.CompilerParams(use_tc_tiling_on_sc=True),
        scratch_types=[
            pltpu.VMEM((CHUNK,), jnp.int32),
            pltpu.VMEM((CHUNK,), jnp.int32),
            pltpu.VMEM((CHUNK,), jnp.int32),
            pltpu.VMEM((CHUNK, ENT_DIM), jnp.float32),
            pltpu.VMEM((CHUNK, ENT_DIM), jnp.float32),
            pltpu.VMEM((REL_DIM * N_REL,), jnp.float32),
            pltpu.VMEM((CHUNK, OUT_DIM), jnp.float32),
            pltpu.SemaphoreType.DMA,
        ],
    )(_body)
    return k(h.astype(jnp.int32), r.astype(jnp.int32), t.astype(jnp.int32),
             ent_rm, rel_weight.reshape(-1))
